# baseline (speedup 1.0000x reference)
import functools

import jax
import jax.numpy as jnp
import numpy as np
from jax import lax
from jax.experimental import pallas as pl
from jax.experimental.pallas import tpu as pltpu
from jax.experimental.pallas import tpu_sc as plsc

_IDX = (3, 7, 15, 22, 31, 44, 58, 63, 71, 85, 92, 101, 110, 118, 124, 127)

_NF = 200
_NBC = 128


def kernel(inputs):
    n = inputs.shape[0]
    nw = 32
    assert n == nw * _NBC
    idx_arr = jnp.asarray(np.array(_IDX, dtype=np.int32))
    mesh = plsc.VectorSubcoreMesh(core_axis_name="c", subcore_axis_name="s")

    @functools.partial(
        pl.kernel, mesh=mesh,
        out_type=jax.ShapeDtypeStruct((_NF, 16, n), jnp.float32),
        compiler_params=pltpu.CompilerParams(needs_layout_passes=False),
        scratch_types=[
            pltpu.VMEM((16,), jnp.int32),
            pltpu.VMEM((_NBC, 128), jnp.float32),
            pltpu.VMEM((16, _NBC), jnp.float32),
        ],
    )
    def sc_gather(x_hbm, idx_hbm, out_hbm, idx_v, xbuf, obuf):
        wid = lax.axis_index("s") * 2 + lax.axis_index("c")
        b0 = wid * _NBC
        pltpu.sync_copy(idx_hbm, idx_v)
        idx16 = idx_v[...]

        def per_f(f, carry):
            pltpu.sync_copy(x_hbm.at[pl.ds(b0, _NBC), f, :], xbuf)

            def per_b(b, c2):
                bvec = jnp.full((16,), b, dtype=jnp.int32)
                g = plsc.load_gather(xbuf, [bvec, idx16])
                plsc.store_scatter(
                    obuf, [lax.iota(jnp.int32, 16), bvec], g)
                return c2

            lax.fori_loop(0, _NBC, per_b, 0)
            pltpu.sync_copy(obuf, out_hbm.at[f, :, pl.ds(b0, _NBC)])
            return carry

        lax.fori_loop(0, _NF, per_f, 0)

    out_t = sc_gather(inputs, idx_arr)
    return out_t.transpose(2, 0, 1)


# speedup vs baseline: 1.9097x; 1.9097x over previous
import functools

import jax
import jax.numpy as jnp
import numpy as np
from jax import lax
from jax.experimental import pallas as pl
from jax.experimental.pallas import tpu as pltpu
from jax.experimental.pallas import tpu_sc as plsc

_IDX = (3, 7, 15, 22, 31, 44, 58, 63, 71, 85, 92, 101, 110, 118, 124, 127)

_NF = 200
_NBC = 128


def kernel(inputs):
    n = inputs.shape[0]
    nw = 32
    assert n == nw * _NBC
    idx_arr = jnp.asarray(np.array(_IDX, dtype=np.int32))
    mesh = plsc.VectorSubcoreMesh(core_axis_name="c", subcore_axis_name="s")

    @functools.partial(
        pl.kernel, mesh=mesh,
        out_type=jax.ShapeDtypeStruct((_NF, 16, n), jnp.float32),
        compiler_params=pltpu.CompilerParams(needs_layout_passes=False),
        scratch_types=[
            pltpu.VMEM((16,), jnp.int32),
            pltpu.VMEM((2, _NBC, 128), jnp.float32),
            pltpu.VMEM((2, 16, _NBC), jnp.float32),
            pltpu.SemaphoreType.DMA((2,)),
            pltpu.SemaphoreType.DMA((2,)),
        ],
    )
    def sc_gather(x_hbm, idx_hbm, out_hbm, idx_v, xbuf, obuf, isem, osem):
        wid = lax.axis_index("s") * 2 + lax.axis_index("c")
        b0 = wid * _NBC
        pltpu.sync_copy(idx_hbm, idx_v)
        idx16 = idx_v[...]
        iota16 = lax.iota(jnp.int32, 16)

        def in_copy(f):
            slot = lax.rem(f, 2)
            return pltpu.make_async_copy(
                x_hbm.at[pl.ds(b0, _NBC), f, :], xbuf.at[slot], isem.at[slot])

        def out_copy(f):
            slot = lax.rem(f, 2)
            return pltpu.make_async_copy(
                obuf.at[slot], out_hbm.at[f, :, pl.ds(b0, _NBC)],
                osem.at[slot])

        def per_f(f, carry):
            slot = lax.rem(f, 2)

            @pl.when(f == 0)
            def _first():
                in_copy(f).start()

            @pl.when(f + 1 < _NF)
            def _prefetch():
                in_copy(f + 1).start()

            @pl.when(f >= 2)
            def _drain_out():
                out_copy(f - 2).wait()

            in_copy(f).wait()
            slotvec = jnp.full((16,), slot, dtype=jnp.int32)

            def per_b(i, c2):
                for j in range(8):
                    b = i * 8 + j
                    bvec = jnp.full((16,), b, dtype=jnp.int32)
                    g = plsc.load_gather(xbuf, [slotvec, bvec, idx16])
                    plsc.store_scatter(obuf, [slotvec, iota16, bvec], g)
                return c2

            lax.fori_loop(0, _NBC // 8, per_b, 0)
            out_copy(f).start()
            return carry

        lax.fori_loop(0, _NF, per_f, 0)
        out_copy(_NF - 2).wait()
        out_copy(_NF - 1).wait()

    out_t = sc_gather(inputs, idx_arr)
    return out_t.transpose(2, 0, 1)


# speedup vs baseline: 4.2714x; 2.2367x over previous
import jax
import jax.numpy as jnp
import numpy as np
from jax.experimental import pallas as pl
from jax.experimental.pallas import tpu as pltpu

_IDX = (3, 7, 15, 22, 31, 44, 58, 63, 71, 85, 92, 101, 110, 118, 124, 127)

_SEL_T = np.zeros((16, 128), dtype=np.float32)
for _k, _i in enumerate(_IDX):
    _SEL_T[_k, _i] = 1.0

_NF = 200


_DEPTH = 8


def _gather_body(x_hbm, s_ref, o_ref, xs_ref, sem):
    f = pl.program_id(0)

    @pl.when(f == 0)
    def _first():
        for d in range(_DEPTH - 1):
            pltpu.make_async_copy(x_hbm.at[:, d, :], xs_ref.at[d], sem.at[d]).start()

    @pl.when(f + _DEPTH - 1 < _NF)
    def _prefetch():
        nxt = f + _DEPTH - 1
        pltpu.make_async_copy(
            x_hbm.at[:, nxt, :], xs_ref.at[nxt % _DEPTH], sem.at[nxt % _DEPTH]
        ).start()

    pltpu.make_async_copy(
        x_hbm.at[:, f, :], xs_ref.at[f % _DEPTH], sem.at[f % _DEPTH]).wait()
    x = xs_ref[f % _DEPTH]
    g_t = jax.lax.dot_general(
        s_ref[...], x, (((1,), (1,)), ((), ())),
        preferred_element_type=jnp.float32)
    o_ref[...] = g_t.reshape(o_ref.shape)


def kernel(inputs):
    n = inputs.shape[0]
    sel_t = jnp.asarray(_SEL_T)
    out_t = pl.pallas_call(
        _gather_body,
        grid=(_NF,),
        in_specs=[
            pl.BlockSpec(memory_space=pl.ANY),
            pl.BlockSpec((16, 128), lambda f: (0, 0)),
        ],
        out_specs=pl.BlockSpec((1, 16, n), lambda f: (f, 0, 0)),
        out_shape=jax.ShapeDtypeStruct((200, 16, n), inputs.dtype),
        scratch_shapes=[
            pltpu.VMEM((_DEPTH, n, 128), jnp.float32),
            pltpu.SemaphoreType.DMA((_DEPTH,)),
        ],
        compiler_params=pltpu.CompilerParams(
            dimension_semantics=("arbitrary",)),
    )(inputs, sel_t)
    return out_t.transpose(2, 0, 1)
